# baseline (speedup 1.0000x reference)
import functools

import jax
import jax.numpy as jnp
from jax import lax
from jax.experimental import pallas as pl
from jax.experimental.pallas import tpu as pltpu
from jax.experimental.pallas import tpu_sc as plsc

N = 10000
E = 320000
D = 128
L = 4
G = 64

NC = 2
NS = 16
NW = NC * NS
EPW = E // NW
C = 80
NCH = EPW // C
NPAD = 10240
RPS = N // NS
RQ = 624
RT = N - NS * RQ
W = NPAD // NS

@functools.cache
def _mesh():
    return plsc.VectorSubcoreMesh(
        core_axis_name="c", subcore_axis_name="s",
        num_cores=NC, num_subcores=NS)


def _sc_degrees_body(src_ref, dst_ref, out_ref, hist_s, hist_d,
                     src_v, dst_v, ones_v, zbuf):
    cid = lax.axis_index("c")
    sid = lax.axis_index("s")
    wid = cid * NS + sid
    zeros16 = jnp.zeros((16,), jnp.float32)
    ones16 = jnp.ones((16,), jnp.float32)

    def fill(i, _):
        zbuf[pl.ds(i * 16, 16)] = zeros16
        return _

    lax.fori_loop(0, W // 16, fill, None)

    def fill1(i, _):
        ones_v[pl.ds(i * 16, 16)] = ones16
        return _

    lax.fori_loop(0, C // 16, fill1, None)

    pltpu.sync_copy(zbuf, hist_s.at[pl.ds(sid * W, W)])
    pltpu.sync_copy(zbuf, hist_d.at[pl.ds(sid * W, W)])
    pltpu.sync_copy(src_ref.at[wid], src_v)
    pltpu.sync_copy(dst_ref.at[wid], dst_v)
    plsc.subcore_barrier()

    def body(j, _):
        pltpu.sync_copy(ones_v, hist_s.at[src_v.at[j]], add=True)
        pltpu.sync_copy(ones_v, hist_d.at[dst_v.at[j]], add=True)
        return _

    lax.fori_loop(0, NCH, body, None)
    plsc.subcore_barrier()
    pltpu.sync_copy(hist_s.at[pl.ds(sid * W, W)],
                    out_ref.at[cid, 0, pl.ds(sid * W, W)])
    pltpu.sync_copy(hist_d.at[pl.ds(sid * W, W)],
                    out_ref.at[cid, 1, pl.ds(sid * W, W)])


@functools.cache
def _build_sc_degrees():
    return pl.kernel(
        _sc_degrees_body,
        out_type=jax.ShapeDtypeStruct((NC, 2, NPAD), jnp.float32),
        mesh=_mesh(),
        scratch_types=[
            pltpu.VMEM_SHARED((NPAD,), jnp.float32),
            pltpu.VMEM_SHARED((NPAD,), jnp.float32),
            pltpu.VMEM((NCH, C), jnp.int32),
            pltpu.VMEM((NCH, C), jnp.int32),
            pltpu.VMEM((C,), jnp.float32),
            pltpu.VMEM((W,), jnp.float32),
        ],
    )


def _sc_degrees(src_r, dst_r):
    return _build_sc_degrees()(src_r, dst_r)


NBUF = 3
IDXR = 6


def _sc_aggregate_body(hw_ref, ix_ref, zr_ref, out_ref,
                       acc, idxb, rows, isem, gsem, ssem):
    cid = lax.axis_index("c")
    sid = lax.axis_index("s")
    wid = cid * NS + sid
    pltpu.sync_copy(zr_ref.at[pl.ds(0, RQ)], acc.at[pl.ds(sid * RQ, RQ)])

    @pl.when(sid == NS - 1)
    def _():
        pltpu.sync_copy(zr_ref.at[pl.ds(0, RT)], acc.at[pl.ds(NS * RQ, RT)])

    plsc.subcore_barrier()

    def i_start(k, b):
        pltpu.async_copy(ix_ref.at[wid, k], idxb.at[b], isem.at[b])

    def i_wait(k, b):
        pltpu.make_async_copy(ix_ref.at[wid, k], idxb.at[b],
                              isem.at[b]).wait()

    def g_start(j, rb, ib):
        pltpu.async_copy(hw_ref.at[idxb.at[ib, 0]], rows.at[rb],
                         gsem.at[rb])

    def g_wait(j, rb, ib):
        pltpu.make_async_copy(hw_ref.at[idxb.at[ib, 0]], rows.at[rb],
                              gsem.at[rb]).wait()

    def s_start(j, rb, ib):
        pltpu.async_copy(rows.at[rb], acc.at[idxb.at[ib, 1]], ssem.at[rb],
                         add=True)

    def s_wait(j, rb, ib):
        pltpu.make_async_copy(rows.at[rb], acc.at[idxb.at[ib, 1]],
                              ssem.at[rb]).wait()

    for k in range(4):
        i_start(k, k)
    for k in range(NBUF - 1):
        i_wait(k, k)
        g_start(k, k, k)

    def body(j, _):
        rb = lax.rem(j, NBUF)
        ib = lax.rem(j, IDXR)
        rbn = lax.rem(j + NBUF - 1, NBUF)
        ibn = lax.rem(j + NBUF - 1, IDXR)

        @pl.when(j >= 1)
        def _():
            s_wait(j - 1, rbn, lax.rem(j + IDXR - 1, IDXR))

        @pl.when(j + NBUF - 1 < NCH)
        def _():
            i_wait(j + NBUF - 1, ibn)
            g_start(j + NBUF - 1, rbn, ibn)

        @pl.when(j + 4 < NCH)
        def _():
            i_start(j + 4, lax.rem(j + 4, IDXR))

        g_wait(j, rb, ib)
        s_start(j, rb, ib)
        return _

    lax.fori_loop(0, NCH, body, None)
    s_wait(NCH - 1, (NCH - 1) % NBUF, (NCH - 1) % IDXR)
    plsc.subcore_barrier()
    pltpu.sync_copy(acc.at[pl.ds(sid * RQ, RQ)],
                    out_ref.at[cid, pl.ds(sid * RQ, RQ)])

    @pl.when(sid == NS - 1)
    def _():
        pltpu.sync_copy(acc.at[pl.ds(NS * RQ, RT)],
                        out_ref.at[cid, pl.ds(NS * RQ, RT)])


@functools.cache
def _build_sc_aggregate():
    return pl.kernel(
        _sc_aggregate_body,
        out_type=jax.ShapeDtypeStruct((NC, N, D), jnp.float32),
        mesh=_mesh(),
        scratch_types=[
            pltpu.VMEM_SHARED((N, D), jnp.float32),
            pltpu.VMEM((IDXR, 2, C), jnp.int32),
            pltpu.VMEM((NBUF, C, D), jnp.float32),
            pltpu.SemaphoreType.DMA((IDXR,)),
            pltpu.SemaphoreType.DMA((NBUF,)),
            pltpu.SemaphoreType.DMA((NBUF,)),
        ],
    )


def _sc_aggregate(hw, ix, zr):
    return _build_sc_aggregate()(hw, ix, zr)


BN = 2000
NG = N // BN


def _mm0_body(x_ref, w_ref, ns_ref, o_ref):
    o_ref[...] = jnp.dot(x_ref[...], w_ref[...],
                         preferred_element_type=jnp.float32) * ns_ref[...]


def _mm0(x, w, ns):
    return pl.pallas_call(
        _mm0_body,
        grid=(NG,),
        in_specs=[
            pl.BlockSpec((BN, D), lambda i: (i, 0)),
            pl.BlockSpec((D, D), lambda i: (0, 0)),
            pl.BlockSpec((BN, 1), lambda i: (i, 0)),
        ],
        out_specs=pl.BlockSpec((BN, D), lambda i: (i, 0)),
        out_shape=jax.ShapeDtypeStruct((N, D), jnp.float32),
    )(x, w, ns)


def _phase_a(l, agg_ref, nd_ref, sn_ref, b_ref, t_v, s1, s2, i):
    @pl.when(i == 0)
    def _():
        s1[...] = jnp.zeros_like(s1)
        s2[...] = jnp.zeros_like(s2)

    @pl.when(i < NG)
    def _():
        a = agg_ref[0] + agg_ref[1]
        t = (a * nd_ref[...] + b_ref[l:l + 1, :]) * sn_ref[...]
        t_v[pl.ds(i * BN, BN), :] = t
        s1[...] += jnp.sum(t, axis=0, keepdims=True)
        s2[...] += jnp.sum(t * t, axis=0, keepdims=True)


def _bn_apply(l, t_v, s1, s2, gm_ref, bt_ref, hin_ref, i):
    g = i - NG
    t = t_v[pl.ds(g * BN, BN), :]
    mu = s1[...] / N
    ig = gm_ref[l:l + 1, :] * lax.rsqrt(s2[...] / N - mu * mu + 1e-5)
    return hin_ref[...] + jnp.maximum((t - mu) * ig + bt_ref[l:l + 1, :],
                                      0.0)


def _layer_body(l, agg_ref, nd_ref, sn_ref, b_ref, gm_ref, bt_ref, hin_ref,
                w_ref, ns_ref, ho_ref, hw_ref, t_v, s1, s2):
    i = pl.program_id(0)
    _phase_a(l, agg_ref, nd_ref, sn_ref, b_ref, t_v, s1, s2, i)

    @pl.when(i >= NG)
    def _():
        hn = _bn_apply(l, t_v, s1, s2, gm_ref, bt_ref, hin_ref, i)
        ho_ref[...] = hn
        hw_ref[...] = jnp.dot(hn, w_ref[0],
                              preferred_element_type=jnp.float32) * ns_ref[...]


def _layer(l, aggp, nd, sn, h, Ws, bs, gammas, betas, ns):
    pa = lambda i: jnp.minimum(i, NG - 1)
    pb = lambda i: jnp.maximum(i - NG, 0)
    return pl.pallas_call(
        functools.partial(_layer_body, l),
        grid=(2 * NG,),
        in_specs=[
            pl.BlockSpec((NC, BN, D), lambda i: (0, pa(i), 0)),
            pl.BlockSpec((BN, 1), lambda i: (pa(i), 0)),
            pl.BlockSpec((BN, 1), lambda i: (pa(i), 0)),
            pl.BlockSpec((L, D), lambda i: (0, 0)),
            pl.BlockSpec((L, D), lambda i: (0, 0)),
            pl.BlockSpec((L, D), lambda i: (0, 0)),
            pl.BlockSpec((BN, D), lambda i: (pb(i), 0)),
            pl.BlockSpec((1, D, D), lambda i: (l + 1, 0, 0)),
            pl.BlockSpec((BN, 1), lambda i: (pb(i), 0)),
        ],
        out_specs=[
            pl.BlockSpec((BN, D), lambda i: (pb(i), 0)),
            pl.BlockSpec((BN, D), lambda i: (pb(i), 0)),
        ],
        out_shape=[
            jax.ShapeDtypeStruct((N, D), jnp.float32),
            jax.ShapeDtypeStruct((N, D), jnp.float32),
        ],
        scratch_shapes=[
            pltpu.VMEM((N, D), jnp.float32),
            pltpu.VMEM((1, D), jnp.float32),
            pltpu.VMEM((1, D), jnp.float32),
        ],
    )(aggp, nd, sn, bs, gammas, betas, h, Ws, ns)


def _final_body(l, agg_ref, nd_ref, sn_ref, b_ref, gm_ref, bt_ref, hin_ref,
                g_ref, o_ref, t_v, s1, s2, accg, cntg):
    i = pl.program_id(0)
    _phase_a(l, agg_ref, nd_ref, sn_ref, b_ref, t_v, s1, s2, i)

    @pl.when(i >= NG)
    def _():
        hn = _bn_apply(l, t_v, s1, s2, gm_ref, bt_ref, hin_ref, i)
        oh = (g_ref[...] == lax.broadcasted_iota(jnp.int32, (BN, G), 1)
              ).astype(jnp.float32)

        @pl.when(i == NG)
        def _():
            accg[...] = jnp.zeros_like(accg)
            cntg[...] = jnp.zeros_like(cntg)

        dnums = (((0,), (0,)), ((), ()))
        accg[...] += lax.dot_general(oh, hn, dnums,
                                     preferred_element_type=jnp.float32)
        cntg[...] += lax.dot_general(oh, jnp.ones((BN, D), jnp.float32),
                                     dnums,
                                     preferred_element_type=jnp.float32)

        @pl.when(i == 2 * NG - 1)
        def _():
            o_ref[...] = accg[...] / jnp.maximum(cntg[...], 1.0)


def _layer_final(l, aggp, nd, sn, h, bs, gammas, betas, gids):
    pa = lambda i: jnp.minimum(i, NG - 1)
    pb = lambda i: jnp.maximum(i - NG, 0)
    return pl.pallas_call(
        functools.partial(_final_body, l),
        grid=(2 * NG,),
        in_specs=[
            pl.BlockSpec((NC, BN, D), lambda i: (0, pa(i), 0)),
            pl.BlockSpec((BN, 1), lambda i: (pa(i), 0)),
            pl.BlockSpec((BN, 1), lambda i: (pa(i), 0)),
            pl.BlockSpec((L, D), lambda i: (0, 0)),
            pl.BlockSpec((L, D), lambda i: (0, 0)),
            pl.BlockSpec((L, D), lambda i: (0, 0)),
            pl.BlockSpec((BN, D), lambda i: (pb(i), 0)),
            pl.BlockSpec((BN, 1), lambda i: (pb(i), 0)),
        ],
        out_specs=pl.BlockSpec((G, D), lambda i: (0, 0)),
        out_shape=jax.ShapeDtypeStruct((G, D), jnp.float32),
        scratch_shapes=[
            pltpu.VMEM((N, D), jnp.float32),
            pltpu.VMEM((1, D), jnp.float32),
            pltpu.VMEM((1, D), jnp.float32),
            pltpu.VMEM((G, D), jnp.float32),
            pltpu.VMEM((G, D), jnp.float32),
        ],
    )(aggp, nd, sn, bs, gammas, betas, h, gids)


def kernel(nodes_feat, edge_index, nodes_num_norm_sqrt, graph_ids,
           Ws, bs, gammas, betas):
    f32 = jnp.float32
    src_r = edge_index[0].reshape(NW, NCH, C)
    dst_r = edge_index[1].reshape(NW, NCH, C)
    idx2 = jnp.stack([src_r, dst_r], axis=2)
    zr = jnp.zeros((RPS, D), f32)

    deg = _sc_degrees(src_r, dst_r)
    degs = deg[0] + deg[1]
    norm_src = lax.rsqrt(jnp.maximum(degs[0, :N], 1.0)).reshape(N, 1)
    norm_dst = lax.rsqrt(jnp.maximum(degs[1, :N], 1.0)).reshape(N, 1)
    gids_col = graph_ids.reshape(N, 1)

    h = nodes_feat
    hw = _mm0(nodes_feat, Ws[0], norm_src)
    out = None
    for l in range(L):
        aggp = _sc_aggregate(hw, idx2, zr)
        if l < L - 1:
            h, hw = _layer(l, aggp, norm_dst, nodes_num_norm_sqrt, h,
                           Ws, bs, gammas, betas, norm_src)
        else:
            out = _layer_final(l, aggp, norm_dst, nodes_num_norm_sqrt, h,
                               bs, gammas, betas, gids_col)
    return out


# speedup vs baseline: 1.0136x; 1.0136x over previous
import functools

import jax
import jax.numpy as jnp
from jax import lax
from jax.experimental import pallas as pl
from jax.experimental.pallas import tpu as pltpu
from jax.experimental.pallas import tpu_sc as plsc

N = 10000
E = 320000
D = 128
L = 4
G = 64

NC = 2
NS = 16
NW = NC * NS
EPW = E // NW
C = 80
NCH = EPW // C
NPAD = 10240
RPS = N // NS
RQ = 624
RT = N - NS * RQ
W = NPAD // NS

@functools.cache
def _mesh():
    return plsc.VectorSubcoreMesh(
        core_axis_name="c", subcore_axis_name="s",
        num_cores=NC, num_subcores=NS)


DR = 4


def _sc_degrees_body(src_ref, dst_ref, out_ref, hist_s, hist_d,
                     src_v, dst_v, ones_v, zbuf, sems, semd):
    cid = lax.axis_index("c")
    sid = lax.axis_index("s")
    wid = cid * NS + sid
    zeros16 = jnp.zeros((16,), jnp.float32)
    ones16 = jnp.ones((16,), jnp.float32)

    def fill(i, _):
        zbuf[pl.ds(i * 16, 16)] = zeros16
        return _

    lax.fori_loop(0, W // 16, fill, None)

    def fill1(i, _):
        ones_v[pl.ds(i * 16, 16)] = ones16
        return _

    lax.fori_loop(0, C // 16, fill1, None)

    pltpu.sync_copy(zbuf, hist_s.at[pl.ds(sid * W, W)])
    pltpu.sync_copy(zbuf, hist_d.at[pl.ds(sid * W, W)])
    pltpu.sync_copy(src_ref.at[wid], src_v)
    pltpu.sync_copy(dst_ref.at[wid], dst_v)
    plsc.subcore_barrier()

    def body(j, _):
        bm = lax.rem(j, DR)

        @pl.when(j >= DR)
        def _():
            pltpu.make_async_copy(ones_v, hist_s.at[src_v.at[j - DR]],
                                  sems.at[bm]).wait()
            pltpu.make_async_copy(ones_v, hist_d.at[dst_v.at[j - DR]],
                                  semd.at[bm]).wait()

        pltpu.async_copy(ones_v, hist_s.at[src_v.at[j]], sems.at[bm],
                         add=True)
        pltpu.async_copy(ones_v, hist_d.at[dst_v.at[j]], semd.at[bm],
                         add=True)
        return _

    lax.fori_loop(0, NCH, body, None)
    for k in range(NCH - DR, NCH):
        pltpu.make_async_copy(ones_v, hist_s.at[src_v.at[k]],
                              sems.at[k % DR]).wait()
        pltpu.make_async_copy(ones_v, hist_d.at[dst_v.at[k]],
                              semd.at[k % DR]).wait()
    plsc.subcore_barrier()
    pltpu.sync_copy(hist_s.at[pl.ds(sid * W, W)],
                    out_ref.at[cid, 0, pl.ds(sid * W, W)])
    pltpu.sync_copy(hist_d.at[pl.ds(sid * W, W)],
                    out_ref.at[cid, 1, pl.ds(sid * W, W)])


@functools.cache
def _build_sc_degrees():
    return pl.kernel(
        _sc_degrees_body,
        out_type=jax.ShapeDtypeStruct((NC, 2, NPAD), jnp.float32),
        mesh=_mesh(),
        scratch_types=[
            pltpu.VMEM_SHARED((NPAD,), jnp.float32),
            pltpu.VMEM_SHARED((NPAD,), jnp.float32),
            pltpu.VMEM((NCH, C), jnp.int32),
            pltpu.VMEM((NCH, C), jnp.int32),
            pltpu.VMEM((C,), jnp.float32),
            pltpu.VMEM((W,), jnp.float32),
            pltpu.SemaphoreType.DMA((DR,)),
            pltpu.SemaphoreType.DMA((DR,)),
        ],
    )


def _sc_degrees(src_r, dst_r):
    return _build_sc_degrees()(src_r, dst_r)


NBUF = 3
IDXR = 6


def _sc_aggregate_body(hw_ref, ix_ref, zr_ref, out_ref,
                       acc, idxb, rows, isem, gsem, ssem):
    cid = lax.axis_index("c")
    sid = lax.axis_index("s")
    wid = cid * NS + sid
    pltpu.sync_copy(zr_ref.at[pl.ds(0, RQ)], acc.at[pl.ds(sid * RQ, RQ)])

    @pl.when(sid == NS - 1)
    def _():
        pltpu.sync_copy(zr_ref.at[pl.ds(0, RT)], acc.at[pl.ds(NS * RQ, RT)])

    plsc.subcore_barrier()

    def i_start(k, b):
        pltpu.async_copy(ix_ref.at[wid, k], idxb.at[b], isem.at[b])

    def i_wait(k, b):
        pltpu.make_async_copy(ix_ref.at[wid, k], idxb.at[b],
                              isem.at[b]).wait()

    def g_start(j, rb, ib):
        pltpu.async_copy(hw_ref.at[idxb.at[ib, 0]], rows.at[rb],
                         gsem.at[rb])

    def g_wait(j, rb, ib):
        pltpu.make_async_copy(hw_ref.at[idxb.at[ib, 0]], rows.at[rb],
                              gsem.at[rb]).wait()

    def s_start(j, rb, ib):
        pltpu.async_copy(rows.at[rb], acc.at[idxb.at[ib, 1]], ssem.at[rb],
                         add=True)

    def s_wait(j, rb, ib):
        pltpu.make_async_copy(rows.at[rb], acc.at[idxb.at[ib, 1]],
                              ssem.at[rb]).wait()

    for k in range(4):
        i_start(k, k)
    for k in range(NBUF - 1):
        i_wait(k, k)
        g_start(k, k, k)

    def body(j, _):
        rb = lax.rem(j, NBUF)
        ib = lax.rem(j, IDXR)
        rbn = lax.rem(j + NBUF - 1, NBUF)
        ibn = lax.rem(j + NBUF - 1, IDXR)

        @pl.when(j >= 1)
        def _():
            s_wait(j - 1, rbn, lax.rem(j + IDXR - 1, IDXR))

        @pl.when(j + NBUF - 1 < NCH)
        def _():
            i_wait(j + NBUF - 1, ibn)
            g_start(j + NBUF - 1, rbn, ibn)

        @pl.when(j + 4 < NCH)
        def _():
            i_start(j + 4, lax.rem(j + 4, IDXR))

        g_wait(j, rb, ib)
        s_start(j, rb, ib)
        return _

    lax.fori_loop(0, NCH, body, None)
    s_wait(NCH - 1, (NCH - 1) % NBUF, (NCH - 1) % IDXR)
    plsc.subcore_barrier()
    pltpu.sync_copy(acc.at[pl.ds(sid * RQ, RQ)],
                    out_ref.at[cid, pl.ds(sid * RQ, RQ)])

    @pl.when(sid == NS - 1)
    def _():
        pltpu.sync_copy(acc.at[pl.ds(NS * RQ, RT)],
                        out_ref.at[cid, pl.ds(NS * RQ, RT)])


@functools.cache
def _build_sc_aggregate():
    return pl.kernel(
        _sc_aggregate_body,
        out_type=jax.ShapeDtypeStruct((NC, N, D), jnp.float32),
        mesh=_mesh(),
        scratch_types=[
            pltpu.VMEM_SHARED((N, D), jnp.float32),
            pltpu.VMEM((IDXR, 2, C), jnp.int32),
            pltpu.VMEM((NBUF, C, D), jnp.float32),
            pltpu.SemaphoreType.DMA((IDXR,)),
            pltpu.SemaphoreType.DMA((NBUF,)),
            pltpu.SemaphoreType.DMA((NBUF,)),
        ],
    )


def _sc_aggregate(hw, ix, zr):
    return _build_sc_aggregate()(hw, ix, zr)


BN = 2000
NG = N // BN


def _mm0_body(x_ref, w_ref, ns_ref, o_ref):
    o_ref[...] = jnp.dot(x_ref[...], w_ref[...],
                         preferred_element_type=jnp.float32) * ns_ref[...]


def _mm0(x, w, ns):
    return pl.pallas_call(
        _mm0_body,
        grid=(NG,),
        in_specs=[
            pl.BlockSpec((BN, D), lambda i: (i, 0)),
            pl.BlockSpec((D, D), lambda i: (0, 0)),
            pl.BlockSpec((BN, 1), lambda i: (i, 0)),
        ],
        out_specs=pl.BlockSpec((BN, D), lambda i: (i, 0)),
        out_shape=jax.ShapeDtypeStruct((N, D), jnp.float32),
    )(x, w, ns)


def _phase_a(l, agg_ref, nd_ref, sn_ref, b_ref, t_v, s1, s2, i):
    @pl.when(i == 0)
    def _():
        s1[...] = jnp.zeros_like(s1)
        s2[...] = jnp.zeros_like(s2)

    @pl.when(i < NG)
    def _():
        a = agg_ref[0] + agg_ref[1]
        t = (a * nd_ref[...] + b_ref[l:l + 1, :]) * sn_ref[...]
        t_v[pl.ds(i * BN, BN), :] = t
        s1[...] += jnp.sum(t, axis=0, keepdims=True)
        s2[...] += jnp.sum(t * t, axis=0, keepdims=True)


def _bn_apply(l, t_v, s1, s2, gm_ref, bt_ref, hin_ref, i):
    g = i - NG
    t = t_v[pl.ds(g * BN, BN), :]
    mu = s1[...] / N
    ig = gm_ref[l:l + 1, :] * lax.rsqrt(s2[...] / N - mu * mu + 1e-5)
    return hin_ref[...] + jnp.maximum((t - mu) * ig + bt_ref[l:l + 1, :],
                                      0.0)


def _layer_body(l, agg_ref, nd_ref, sn_ref, b_ref, gm_ref, bt_ref, hin_ref,
                w_ref, ns_ref, ho_ref, hw_ref, t_v, s1, s2):
    i = pl.program_id(0)
    _phase_a(l, agg_ref, nd_ref, sn_ref, b_ref, t_v, s1, s2, i)

    @pl.when(i >= NG)
    def _():
        hn = _bn_apply(l, t_v, s1, s2, gm_ref, bt_ref, hin_ref, i)
        ho_ref[...] = hn
        hw_ref[...] = jnp.dot(hn, w_ref[0],
                              preferred_element_type=jnp.float32) * ns_ref[...]


def _layer(l, aggp, nd, sn, h, Ws, bs, gammas, betas, ns):
    pa = lambda i: jnp.minimum(i, NG - 1)
    pb = lambda i: jnp.maximum(i - NG, 0)
    return pl.pallas_call(
        functools.partial(_layer_body, l),
        grid=(2 * NG,),
        in_specs=[
            pl.BlockSpec((NC, BN, D), lambda i: (0, pa(i), 0)),
            pl.BlockSpec((BN, 1), lambda i: (pa(i), 0)),
            pl.BlockSpec((BN, 1), lambda i: (pa(i), 0)),
            pl.BlockSpec((L, D), lambda i: (0, 0)),
            pl.BlockSpec((L, D), lambda i: (0, 0)),
            pl.BlockSpec((L, D), lambda i: (0, 0)),
            pl.BlockSpec((BN, D), lambda i: (pb(i), 0)),
            pl.BlockSpec((1, D, D), lambda i: (l + 1, 0, 0)),
            pl.BlockSpec((BN, 1), lambda i: (pb(i), 0)),
        ],
        out_specs=[
            pl.BlockSpec((BN, D), lambda i: (pb(i), 0)),
            pl.BlockSpec((BN, D), lambda i: (pb(i), 0)),
        ],
        out_shape=[
            jax.ShapeDtypeStruct((N, D), jnp.float32),
            jax.ShapeDtypeStruct((N, D), jnp.float32),
        ],
        scratch_shapes=[
            pltpu.VMEM((N, D), jnp.float32),
            pltpu.VMEM((1, D), jnp.float32),
            pltpu.VMEM((1, D), jnp.float32),
        ],
    )(aggp, nd, sn, bs, gammas, betas, h, Ws, ns)


def _final_body(l, agg_ref, nd_ref, sn_ref, b_ref, gm_ref, bt_ref, hin_ref,
                g_ref, o_ref, t_v, s1, s2, accg, cntg):
    i = pl.program_id(0)
    _phase_a(l, agg_ref, nd_ref, sn_ref, b_ref, t_v, s1, s2, i)

    @pl.when(i >= NG)
    def _():
        hn = _bn_apply(l, t_v, s1, s2, gm_ref, bt_ref, hin_ref, i)
        oh = (g_ref[...] == lax.broadcasted_iota(jnp.int32, (BN, G), 1)
              ).astype(jnp.float32)

        @pl.when(i == NG)
        def _():
            accg[...] = jnp.zeros_like(accg)
            cntg[...] = jnp.zeros_like(cntg)

        dnums = (((0,), (0,)), ((), ()))
        accg[...] += lax.dot_general(oh, hn, dnums,
                                     preferred_element_type=jnp.float32)
        cntg[...] += lax.dot_general(oh, jnp.ones((BN, D), jnp.float32),
                                     dnums,
                                     preferred_element_type=jnp.float32)

        @pl.when(i == 2 * NG - 1)
        def _():
            o_ref[...] = accg[...] / jnp.maximum(cntg[...], 1.0)


def _layer_final(l, aggp, nd, sn, h, bs, gammas, betas, gids):
    pa = lambda i: jnp.minimum(i, NG - 1)
    pb = lambda i: jnp.maximum(i - NG, 0)
    return pl.pallas_call(
        functools.partial(_final_body, l),
        grid=(2 * NG,),
        in_specs=[
            pl.BlockSpec((NC, BN, D), lambda i: (0, pa(i), 0)),
            pl.BlockSpec((BN, 1), lambda i: (pa(i), 0)),
            pl.BlockSpec((BN, 1), lambda i: (pa(i), 0)),
            pl.BlockSpec((L, D), lambda i: (0, 0)),
            pl.BlockSpec((L, D), lambda i: (0, 0)),
            pl.BlockSpec((L, D), lambda i: (0, 0)),
            pl.BlockSpec((BN, D), lambda i: (pb(i), 0)),
            pl.BlockSpec((BN, 1), lambda i: (pb(i), 0)),
        ],
        out_specs=pl.BlockSpec((G, D), lambda i: (0, 0)),
        out_shape=jax.ShapeDtypeStruct((G, D), jnp.float32),
        scratch_shapes=[
            pltpu.VMEM((N, D), jnp.float32),
            pltpu.VMEM((1, D), jnp.float32),
            pltpu.VMEM((1, D), jnp.float32),
            pltpu.VMEM((G, D), jnp.float32),
            pltpu.VMEM((G, D), jnp.float32),
        ],
    )(aggp, nd, sn, bs, gammas, betas, h, gids)


def kernel(nodes_feat, edge_index, nodes_num_norm_sqrt, graph_ids,
           Ws, bs, gammas, betas):
    f32 = jnp.float32
    src_r = edge_index[0].reshape(NW, NCH, C)
    dst_r = edge_index[1].reshape(NW, NCH, C)
    idx2 = jnp.stack([src_r, dst_r], axis=2)
    zr = jnp.zeros((RPS, D), f32)

    deg = _sc_degrees(src_r, dst_r)
    degs = deg[0] + deg[1]
    norm_src = lax.rsqrt(jnp.maximum(degs[0, :N], 1.0)).reshape(N, 1)
    norm_dst = lax.rsqrt(jnp.maximum(degs[1, :N], 1.0)).reshape(N, 1)
    gids_col = graph_ids.reshape(N, 1)

    h = nodes_feat
    hw = _mm0(nodes_feat, Ws[0], norm_src)
    out = None
    for l in range(L):
        aggp = _sc_aggregate(hw, idx2, zr)
        if l < L - 1:
            h, hw = _layer(l, aggp, norm_dst, nodes_num_norm_sqrt, h,
                           Ws, bs, gammas, betas, norm_src)
        else:
            out = _layer_final(l, aggp, norm_dst, nodes_num_norm_sqrt, h,
                               bs, gammas, betas, gids_col)
    return out


# speedup vs baseline: 1.0293x; 1.0155x over previous
import functools

import jax
import jax.numpy as jnp
from jax import lax
from jax.experimental import pallas as pl
from jax.experimental.pallas import tpu as pltpu
from jax.experimental.pallas import tpu_sc as plsc

N = 10000
E = 320000
D = 128
L = 4
G = 64

NC = 2
NS = 16
NW = NC * NS
EPW = E // NW
C = 80
NCH = EPW // C
NPAD = 10240
RPS = N // NS
RQ = 624
RT = N - NS * RQ
W = NPAD // NS

@functools.cache
def _mesh():
    return plsc.VectorSubcoreMesh(
        core_axis_name="c", subcore_axis_name="s",
        num_cores=NC, num_subcores=NS)


DR = 4


def _sc_degrees_body(src_ref, dst_ref, out_ref, hist_s, hist_d,
                     src_v, dst_v, ones_v, zbuf, sems, semd):
    cid = lax.axis_index("c")
    sid = lax.axis_index("s")
    wid = cid * NS + sid
    zeros16 = jnp.zeros((16,), jnp.float32)
    ones16 = jnp.ones((16,), jnp.float32)

    def fill(i, _):
        zbuf[pl.ds(i * 16, 16)] = zeros16
        return _

    lax.fori_loop(0, W // 16, fill, None)

    def fill1(i, _):
        ones_v[pl.ds(i * 16, 16)] = ones16
        return _

    lax.fori_loop(0, C // 16, fill1, None)

    pltpu.sync_copy(zbuf, hist_s.at[pl.ds(sid * W, W)])
    pltpu.sync_copy(zbuf, hist_d.at[pl.ds(sid * W, W)])
    pltpu.sync_copy(src_ref.at[wid], src_v)
    pltpu.sync_copy(dst_ref.at[wid], dst_v)
    plsc.subcore_barrier()

    def body(j, _):
        bm = lax.rem(j, DR)

        @pl.when(j >= DR)
        def _():
            pltpu.make_async_copy(ones_v, hist_s.at[src_v.at[j - DR]],
                                  sems.at[bm]).wait()
            pltpu.make_async_copy(ones_v, hist_d.at[dst_v.at[j - DR]],
                                  semd.at[bm]).wait()

        pltpu.async_copy(ones_v, hist_s.at[src_v.at[j]], sems.at[bm],
                         add=True)
        pltpu.async_copy(ones_v, hist_d.at[dst_v.at[j]], semd.at[bm],
                         add=True)
        return _

    lax.fori_loop(0, NCH, body, None)
    for k in range(NCH - DR, NCH):
        pltpu.make_async_copy(ones_v, hist_s.at[src_v.at[k]],
                              sems.at[k % DR]).wait()
        pltpu.make_async_copy(ones_v, hist_d.at[dst_v.at[k]],
                              semd.at[k % DR]).wait()
    plsc.subcore_barrier()
    pltpu.sync_copy(hist_s.at[pl.ds(sid * W, W)],
                    out_ref.at[cid, 0, pl.ds(sid * W, W)])
    pltpu.sync_copy(hist_d.at[pl.ds(sid * W, W)],
                    out_ref.at[cid, 1, pl.ds(sid * W, W)])


@functools.cache
def _build_sc_degrees():
    return pl.kernel(
        _sc_degrees_body,
        out_type=jax.ShapeDtypeStruct((NC, 2, NPAD), jnp.float32),
        mesh=_mesh(),
        scratch_types=[
            pltpu.VMEM_SHARED((NPAD,), jnp.float32),
            pltpu.VMEM_SHARED((NPAD,), jnp.float32),
            pltpu.VMEM((NCH, C), jnp.int32),
            pltpu.VMEM((NCH, C), jnp.int32),
            pltpu.VMEM((C,), jnp.float32),
            pltpu.VMEM((W,), jnp.float32),
            pltpu.SemaphoreType.DMA((DR,)),
            pltpu.SemaphoreType.DMA((DR,)),
        ],
    )


def _sc_degrees(src_r, dst_r):
    return _build_sc_degrees()(src_r, dst_r)


NBUF = 3
IDXR = 6


def _sc_aggregate_body(hw_ref, ix_ref, zr_ref, out_ref,
                       acc, idxb, rows, isem, gsem, ssem):
    cid = lax.axis_index("c")
    sid = lax.axis_index("s")
    wid = cid * NS + sid

    def i_start(k, b):
        pltpu.async_copy(ix_ref.at[wid, k], idxb.at[b], isem.at[b])

    def i_wait(k, b):
        pltpu.make_async_copy(ix_ref.at[wid, k], idxb.at[b],
                              isem.at[b]).wait()

    def g_start(j, rb, ib):
        pltpu.async_copy(hw_ref.at[idxb.at[ib, 0]], rows.at[rb],
                         gsem.at[rb])

    def g_wait(j, rb, ib):
        pltpu.make_async_copy(hw_ref.at[idxb.at[ib, 0]], rows.at[rb],
                              gsem.at[rb]).wait()

    def s_start(j, rb, ib):
        pltpu.async_copy(rows.at[rb], acc.at[idxb.at[ib, 1]], ssem.at[rb],
                         add=True)

    def s_wait(j, rb, ib):
        pltpu.make_async_copy(rows.at[rb], acc.at[idxb.at[ib, 1]],
                              ssem.at[rb]).wait()

    for k in range(4):
        i_start(k, k)
    for k in range(NBUF - 1):
        i_wait(k, k)
        g_start(k, k, k)

    pltpu.sync_copy(zr_ref.at[pl.ds(0, RQ)], acc.at[pl.ds(sid * RQ, RQ)])

    @pl.when(sid == NS - 1)
    def _():
        pltpu.sync_copy(zr_ref.at[pl.ds(0, RT)], acc.at[pl.ds(NS * RQ, RT)])

    plsc.subcore_barrier()

    def body(j, _):
        rb = lax.rem(j, NBUF)
        ib = lax.rem(j, IDXR)
        rbn = lax.rem(j + NBUF - 1, NBUF)
        ibn = lax.rem(j + NBUF - 1, IDXR)

        @pl.when(j >= 1)
        def _():
            s_wait(j - 1, rbn, lax.rem(j + IDXR - 1, IDXR))

        @pl.when(j + NBUF - 1 < NCH)
        def _():
            i_wait(j + NBUF - 1, ibn)
            g_start(j + NBUF - 1, rbn, ibn)

        @pl.when(j + 4 < NCH)
        def _():
            i_start(j + 4, lax.rem(j + 4, IDXR))

        g_wait(j, rb, ib)
        s_start(j, rb, ib)
        return _

    lax.fori_loop(0, NCH, body, None)
    s_wait(NCH - 1, (NCH - 1) % NBUF, (NCH - 1) % IDXR)
    plsc.subcore_barrier()
    pltpu.sync_copy(acc.at[pl.ds(sid * RQ, RQ)],
                    out_ref.at[cid, pl.ds(sid * RQ, RQ)])

    @pl.when(sid == NS - 1)
    def _():
        pltpu.sync_copy(acc.at[pl.ds(NS * RQ, RT)],
                        out_ref.at[cid, pl.ds(NS * RQ, RT)])


@functools.cache
def _build_sc_aggregate():
    return pl.kernel(
        _sc_aggregate_body,
        out_type=jax.ShapeDtypeStruct((NC, N, D), jnp.float32),
        mesh=_mesh(),
        scratch_types=[
            pltpu.VMEM_SHARED((N, D), jnp.float32),
            pltpu.VMEM((IDXR, 2, C), jnp.int32),
            pltpu.VMEM((NBUF, C, D), jnp.float32),
            pltpu.SemaphoreType.DMA((IDXR,)),
            pltpu.SemaphoreType.DMA((NBUF,)),
            pltpu.SemaphoreType.DMA((NBUF,)),
        ],
    )


def _sc_aggregate(hw, ix, zr):
    return _build_sc_aggregate()(hw, ix, zr)


BN = 2000
NG = N // BN


def _mm0_body(x_ref, w_ref, ns_ref, o_ref):
    o_ref[...] = jnp.dot(x_ref[...], w_ref[...],
                         preferred_element_type=jnp.float32) * ns_ref[...]


def _mm0(x, w, ns):
    return pl.pallas_call(
        _mm0_body,
        grid=(NG,),
        in_specs=[
            pl.BlockSpec((BN, D), lambda i: (i, 0)),
            pl.BlockSpec((D, D), lambda i: (0, 0)),
            pl.BlockSpec((BN, 1), lambda i: (i, 0)),
        ],
        out_specs=pl.BlockSpec((BN, D), lambda i: (i, 0)),
        out_shape=jax.ShapeDtypeStruct((N, D), jnp.float32),
    )(x, w, ns)


def _phase_a(l, agg_ref, nd_ref, sn_ref, b_ref, t_v, s1, s2, i):
    @pl.when(i == 0)
    def _():
        s1[...] = jnp.zeros_like(s1)
        s2[...] = jnp.zeros_like(s2)

    @pl.when(i < NG)
    def _():
        a = agg_ref[0] + agg_ref[1]
        t = (a * nd_ref[...] + b_ref[l:l + 1, :]) * sn_ref[...]
        t_v[pl.ds(i * BN, BN), :] = t
        s1[...] += jnp.sum(t, axis=0, keepdims=True)
        s2[...] += jnp.sum(t * t, axis=0, keepdims=True)


def _bn_apply(l, t_v, s1, s2, gm_ref, bt_ref, hin_ref, i):
    g = i - NG
    t = t_v[pl.ds(g * BN, BN), :]
    mu = s1[...] / N
    ig = gm_ref[l:l + 1, :] * lax.rsqrt(s2[...] / N - mu * mu + 1e-5)
    return hin_ref[...] + jnp.maximum((t - mu) * ig + bt_ref[l:l + 1, :],
                                      0.0)


def _layer_body(l, agg_ref, nd_ref, sn_ref, b_ref, gm_ref, bt_ref, hin_ref,
                w_ref, ns_ref, ho_ref, hw_ref, t_v, s1, s2):
    i = pl.program_id(0)
    _phase_a(l, agg_ref, nd_ref, sn_ref, b_ref, t_v, s1, s2, i)

    @pl.when(i >= NG)
    def _():
        hn = _bn_apply(l, t_v, s1, s2, gm_ref, bt_ref, hin_ref, i)
        ho_ref[...] = hn
        hw_ref[...] = jnp.dot(hn, w_ref[0],
                              preferred_element_type=jnp.float32) * ns_ref[...]


def _layer(l, aggp, nd, sn, h, Ws, bs, gammas, betas, ns):
    pa = lambda i: jnp.minimum(i, NG - 1)
    pb = lambda i: jnp.maximum(i - NG, 0)
    return pl.pallas_call(
        functools.partial(_layer_body, l),
        grid=(2 * NG,),
        in_specs=[
            pl.BlockSpec((NC, BN, D), lambda i: (0, pa(i), 0)),
            pl.BlockSpec((BN, 1), lambda i: (pa(i), 0)),
            pl.BlockSpec((BN, 1), lambda i: (pa(i), 0)),
            pl.BlockSpec((L, D), lambda i: (0, 0)),
            pl.BlockSpec((L, D), lambda i: (0, 0)),
            pl.BlockSpec((L, D), lambda i: (0, 0)),
            pl.BlockSpec((BN, D), lambda i: (pb(i), 0)),
            pl.BlockSpec((1, D, D), lambda i: (l + 1, 0, 0)),
            pl.BlockSpec((BN, 1), lambda i: (pb(i), 0)),
        ],
        out_specs=[
            pl.BlockSpec((BN, D), lambda i: (pb(i), 0)),
            pl.BlockSpec((BN, D), lambda i: (pb(i), 0)),
        ],
        out_shape=[
            jax.ShapeDtypeStruct((N, D), jnp.float32),
            jax.ShapeDtypeStruct((N, D), jnp.float32),
        ],
        scratch_shapes=[
            pltpu.VMEM((N, D), jnp.float32),
            pltpu.VMEM((1, D), jnp.float32),
            pltpu.VMEM((1, D), jnp.float32),
        ],
    )(aggp, nd, sn, bs, gammas, betas, h, Ws, ns)


def _final_body(l, agg_ref, nd_ref, sn_ref, b_ref, gm_ref, bt_ref, hin_ref,
                g_ref, o_ref, t_v, s1, s2, accg, cntg):
    i = pl.program_id(0)
    _phase_a(l, agg_ref, nd_ref, sn_ref, b_ref, t_v, s1, s2, i)

    @pl.when(i >= NG)
    def _():
        hn = _bn_apply(l, t_v, s1, s2, gm_ref, bt_ref, hin_ref, i)
        oh = (g_ref[...] == lax.broadcasted_iota(jnp.int32, (BN, G), 1)
              ).astype(jnp.float32)

        @pl.when(i == NG)
        def _():
            accg[...] = jnp.zeros_like(accg)
            cntg[...] = jnp.zeros_like(cntg)

        dnums = (((0,), (0,)), ((), ()))
        accg[...] += lax.dot_general(oh, hn, dnums,
                                     preferred_element_type=jnp.float32)
        cntg[...] += lax.dot_general(oh, jnp.ones((BN, D), jnp.float32),
                                     dnums,
                                     preferred_element_type=jnp.float32)

        @pl.when(i == 2 * NG - 1)
        def _():
            o_ref[...] = accg[...] / jnp.maximum(cntg[...], 1.0)


def _layer_final(l, aggp, nd, sn, h, bs, gammas, betas, gids):
    pa = lambda i: jnp.minimum(i, NG - 1)
    pb = lambda i: jnp.maximum(i - NG, 0)
    return pl.pallas_call(
        functools.partial(_final_body, l),
        grid=(2 * NG,),
        in_specs=[
            pl.BlockSpec((NC, BN, D), lambda i: (0, pa(i), 0)),
            pl.BlockSpec((BN, 1), lambda i: (pa(i), 0)),
            pl.BlockSpec((BN, 1), lambda i: (pa(i), 0)),
            pl.BlockSpec((L, D), lambda i: (0, 0)),
            pl.BlockSpec((L, D), lambda i: (0, 0)),
            pl.BlockSpec((L, D), lambda i: (0, 0)),
            pl.BlockSpec((BN, D), lambda i: (pb(i), 0)),
            pl.BlockSpec((BN, 1), lambda i: (pb(i), 0)),
        ],
        out_specs=pl.BlockSpec((G, D), lambda i: (0, 0)),
        out_shape=jax.ShapeDtypeStruct((G, D), jnp.float32),
        scratch_shapes=[
            pltpu.VMEM((N, D), jnp.float32),
            pltpu.VMEM((1, D), jnp.float32),
            pltpu.VMEM((1, D), jnp.float32),
            pltpu.VMEM((G, D), jnp.float32),
            pltpu.VMEM((G, D), jnp.float32),
        ],
    )(aggp, nd, sn, bs, gammas, betas, h, gids)


def kernel(nodes_feat, edge_index, nodes_num_norm_sqrt, graph_ids,
           Ws, bs, gammas, betas):
    f32 = jnp.float32
    src_r = edge_index[0].reshape(NW, NCH, C)
    dst_r = edge_index[1].reshape(NW, NCH, C)
    idx2 = jnp.stack([src_r, dst_r], axis=2)
    zr = jnp.zeros((RPS, D), f32)

    deg = _sc_degrees(src_r, dst_r)
    degs = deg[0] + deg[1]
    norm_src = lax.rsqrt(jnp.maximum(degs[0, :N], 1.0)).reshape(N, 1)
    norm_dst = lax.rsqrt(jnp.maximum(degs[1, :N], 1.0)).reshape(N, 1)
    gids_col = graph_ids.reshape(N, 1)

    h = nodes_feat
    hw = _mm0(nodes_feat, Ws[0], norm_src)
    out = None
    for l in range(L):
        aggp = _sc_aggregate(hw, idx2, zr)
        if l < L - 1:
            h, hw = _layer(l, aggp, norm_dst, nodes_num_norm_sqrt, h,
                           Ws, bs, gammas, betas, norm_src)
        else:
            out = _layer_final(l, aggp, norm_dst, nodes_num_norm_sqrt, h,
                               bs, gammas, betas, gids_col)
    return out
